# baseline (speedup 1.0000x reference)
import functools

import jax
import jax.numpy as jnp
from jax import lax
from jax.experimental import pallas as pl
from jax.experimental.pallas import tpu as pltpu

_LANES = 128


def _round_up(v, m):
    return -(-v // m) * m


def _gap_head_kernel(x_ref, wt_ref, b_ref, out_ref, fea_ref, acc_ref, *,
                     inv_s, total_s, tile_s, masked):
    s = pl.program_id(1)
    tn, c, _ = x_ref.shape
    c_pad = fea_ref.shape[-1]

    chunk = x_ref[...]
    if masked:
        valid = total_s - s * tile_s
        lane = lax.broadcasted_iota(jnp.int32, chunk.shape, 2)
        chunk = jnp.where(lane < valid, chunk, 0.0)

    @pl.when(s == 0)
    def _():
        acc_ref[...] = chunk

    @pl.when(s != 0)
    def _():
        acc_ref[...] = acc_ref[...] + chunk

    @pl.when(s == pl.num_programs(1) - 1)
    def _():
        col = jnp.sum(acc_ref[...], axis=-1, keepdims=True) * inv_s
        rows = [jnp.transpose(col[n]) for n in range(tn)]
        fea = jnp.concatenate(rows, axis=0)
        if c_pad != c:
            fea_st = jnp.concatenate(
                [fea, jnp.zeros((tn, c_pad - c), jnp.float32)], axis=-1)
        else:
            fea_st = fea
        fea_ref[...] = fea_st.astype(fea_ref.dtype)
        out = jnp.dot(fea, wt_ref[...],
                      preferred_element_type=jnp.float32) + b_ref[...]
        out_ref[...] = out.astype(out_ref.dtype)


def _gap_head(x, weight, bias):
    N, C, D, H, W = x.shape
    S = D * H * W
    out_c = weight.shape[0]
    out_pad = _round_up(out_c, _LANES)
    c_pad = _round_up(C, _LANES)

    tn = min(N, 8)

    ts = min(S, _LANES)
    masked = (S % ts) != 0

    grid = (pl.cdiv(N, tn), pl.cdiv(S, ts))

    x_flat = x.reshape(N, C, S)

    wt = jnp.zeros((C, out_pad), jnp.float32).at[:, :out_c].set(
        weight.T.astype(jnp.float32))
    b2 = jnp.zeros((1, out_pad), jnp.float32).at[:, :out_c].set(
        bias.astype(jnp.float32))

    kernel_fn = functools.partial(
        _gap_head_kernel, inv_s=1.0 / S, total_s=S, tile_s=ts, masked=masked)

    out_p, fea_p = pl.pallas_call(
        kernel_fn,
        out_shape=(
            jax.ShapeDtypeStruct((N, out_pad), x.dtype),
            jax.ShapeDtypeStruct((N, c_pad), x.dtype),
        ),
        grid_spec=pltpu.PrefetchScalarGridSpec(
            num_scalar_prefetch=0,
            grid=grid,
            in_specs=[
                pl.BlockSpec((tn, C, ts), lambda n, s: (n, 0, s)),
                pl.BlockSpec((C, out_pad), lambda n, s: (0, 0)),
                pl.BlockSpec((1, out_pad), lambda n, s: (0, 0)),
            ],
            out_specs=(
                pl.BlockSpec((tn, out_pad), lambda n, s: (n, 0)),
                pl.BlockSpec((tn, c_pad), lambda n, s: (n, 0)),
            ),
            scratch_shapes=[pltpu.VMEM((tn, C, ts), jnp.float32)],
        ),
        compiler_params=pltpu.CompilerParams(
            dimension_semantics=("parallel", "arbitrary"),
            vmem_limit_bytes=48 << 20,
        ),
        cost_estimate=pl.CostEstimate(
            flops=N * C * S + 2 * N * C * out_pad,
            transcendentals=0,
            bytes_accessed=N * C * S * 4 + (C + 1) * out_pad * 4
            + N * (out_pad + c_pad) * 4,
        ),
    )(x_flat, wt, b2)

    return out_p[:, :out_c], fea_p[:, :C]


def kernel(x, weight, bias):
    return _gap_head(x, weight, bias)


# speedup vs baseline: 1.2029x; 1.2029x over previous
import functools

import jax
import jax.numpy as jnp
from jax import lax
from jax.experimental import pallas as pl
from jax.experimental.pallas import tpu as pltpu

_LANES = 128


def _round_up(v, m):
    return -(-v // m) * m


def _gap_head_kernel(x_ref, wt_ref, b_ref, out_ref, fea_ref, *, inv_s):
    tn, c, _ = x_ref.shape
    c_pad = fea_ref.shape[-1]

    col = jnp.sum(x_ref[...], axis=-1, keepdims=True) * inv_s
    rows = [jnp.transpose(col[n]) for n in range(tn)]
    fea = jnp.concatenate(rows, axis=0)
    if c_pad != c:
        fea_st = jnp.concatenate(
            [fea, jnp.zeros((tn, c_pad - c), jnp.float32)], axis=-1)
    else:
        fea_st = fea
    fea_ref[...] = fea_st.astype(fea_ref.dtype)
    out = jnp.dot(fea, wt_ref[...],
                  preferred_element_type=jnp.float32) + b_ref[...]
    out_ref[...] = out.astype(out_ref.dtype)


def _gap_head(x, weight, bias):
    N, C, D, H, W = x.shape
    S = D * H * W
    out_c = weight.shape[0]
    out_pad = _round_up(out_c, _LANES)
    c_pad = _round_up(C, _LANES)

    tn = min(N, 8)
    grid = (pl.cdiv(N, tn),)

    x_flat = x.reshape(N, C, S)

    wt = jnp.zeros((C, out_pad), jnp.float32).at[:, :out_c].set(
        weight.T.astype(jnp.float32))
    b2 = jnp.zeros((1, out_pad), jnp.float32).at[:, :out_c].set(
        bias.astype(jnp.float32))

    kernel_fn = functools.partial(_gap_head_kernel, inv_s=1.0 / S)

    out_p, fea_p = pl.pallas_call(
        kernel_fn,
        out_shape=(
            jax.ShapeDtypeStruct((N, out_pad), x.dtype),
            jax.ShapeDtypeStruct((N, c_pad), x.dtype),
        ),
        grid_spec=pltpu.PrefetchScalarGridSpec(
            num_scalar_prefetch=0,
            grid=grid,
            in_specs=[
                pl.BlockSpec((tn, C, S), lambda n: (n, 0, 0)),
                pl.BlockSpec((C, out_pad), lambda n: (0, 0)),
                pl.BlockSpec((1, out_pad), lambda n: (0, 0)),
            ],
            out_specs=(
                pl.BlockSpec((tn, out_pad), lambda n: (n, 0)),
                pl.BlockSpec((tn, c_pad), lambda n: (n, 0)),
            ),
        ),
        compiler_params=pltpu.CompilerParams(
            dimension_semantics=("parallel",),
            vmem_limit_bytes=48 << 20,
        ),
        cost_estimate=pl.CostEstimate(
            flops=N * C * S + 2 * N * C * out_pad,
            transcendentals=0,
            bytes_accessed=N * C * S * 4 + (C + 1) * out_pad * 4
            + N * (out_pad + c_pad) * 4,
        ),
    )(x_flat, wt, b2)

    return out_p[:, :out_c], fea_p[:, :C]


def kernel(x, weight, bias):
    return _gap_head(x, weight, bias)


# speedup vs baseline: 1.2135x; 1.0088x over previous
import functools

import jax
import jax.numpy as jnp
from jax.experimental import pallas as pl
from jax.experimental.pallas import tpu as pltpu

_LANES = 128
_NSTREAM = 4


def _round_up(v, m):
    return -(-v // m) * m


def _gap_head_kernel(*refs, inv_s):
    x_refs = refs[:_NSTREAM]
    wt_ref, b_ref, out_ref, fea_ref = refs[_NSTREAM:]
    tn = x_refs[0].shape[0]
    c = sum(r.shape[1] for r in x_refs)
    c_pad = fea_ref.shape[-1]

    cols = [jnp.sum(r[...], axis=-1, keepdims=True) * inv_s for r in x_refs]
    rows = []
    for n in range(tn):
        rows.append(jnp.concatenate(
            [jnp.transpose(col[n]) for col in cols], axis=1))
    fea = jnp.concatenate(rows, axis=0)
    if c_pad != c:
        fea_st = jnp.concatenate(
            [fea, jnp.zeros((tn, c_pad - c), jnp.float32)], axis=-1)
    else:
        fea_st = fea
    fea_ref[...] = fea_st.astype(fea_ref.dtype)
    out = jnp.dot(fea, wt_ref[...],
                  preferred_element_type=jnp.float32) + b_ref[...]
    out_ref[...] = out.astype(out_ref.dtype)


def _gap_head(x, weight, bias):
    N, C, D, H, W = x.shape
    S = D * H * W
    out_c = weight.shape[0]
    out_pad = _round_up(out_c, _LANES)
    c_pad = _round_up(C, _LANES)

    tn = min(N, 8)
    grid = (pl.cdiv(N, tn),)

    x_flat = x.reshape(N, C, S)

    wt = jnp.zeros((C, out_pad), jnp.float32).at[:, :out_c].set(
        weight.T.astype(jnp.float32))
    b2 = jnp.zeros((1, out_pad), jnp.float32).at[:, :out_c].set(
        bias.astype(jnp.float32))

    kernel_fn = functools.partial(_gap_head_kernel, inv_s=1.0 / S)

    ck = C // _NSTREAM

    def slice_spec(k):
        return pl.BlockSpec((tn, ck, S), lambda n, k=k: (n, k, 0))

    out_p, fea_p = pl.pallas_call(
        kernel_fn,
        out_shape=(
            jax.ShapeDtypeStruct((N, out_pad), x.dtype),
            jax.ShapeDtypeStruct((N, c_pad), x.dtype),
        ),
        grid_spec=pltpu.PrefetchScalarGridSpec(
            num_scalar_prefetch=0,
            grid=grid,
            in_specs=[slice_spec(k) for k in range(_NSTREAM)] + [
                pl.BlockSpec((C, out_pad), lambda n: (0, 0)),
                pl.BlockSpec((1, out_pad), lambda n: (0, 0)),
            ],
            out_specs=(
                pl.BlockSpec((tn, out_pad), lambda n: (n, 0)),
                pl.BlockSpec((tn, c_pad), lambda n: (n, 0)),
            ),
        ),
        compiler_params=pltpu.CompilerParams(
            dimension_semantics=("parallel",),
            vmem_limit_bytes=48 << 20,
        ),
        cost_estimate=pl.CostEstimate(
            flops=N * C * S + 2 * N * C * out_pad,
            transcendentals=0,
            bytes_accessed=N * C * S * 4 + (C + 1) * out_pad * 4
            + N * (out_pad + c_pad) * 4,
        ),
    )(*([x_flat] * _NSTREAM), wt, b2)

    return out_p[:, :out_c], fea_p[:, :C]


def kernel(x, weight, bias):
    return _gap_head(x, weight, bias)


# speedup vs baseline: 3.8023x; 3.1332x over previous
import functools

import jax
import jax.numpy as jnp
from jax import lax
from jax.experimental import pallas as pl
from jax.experimental.pallas import tpu as pltpu

_NSTREAM = 4


def _gap_head_kernel(*refs, inv_s, nstream):
    x_refs = refs[:nstream]
    w_ref, b_ref, out_ref, fea_ref = refs[nstream:]

    part = jnp.sum(x_refs[0][...], axis=1)
    for r in x_refs[1:]:
        part = part + jnp.sum(r[...], axis=1)
    fea = part * inv_s
    fea_ref[...] = fea.astype(fea_ref.dtype)

    out = lax.dot_general(
        fea, w_ref[...],
        dimension_numbers=(((1,), (1,)), ((), ())),
        preferred_element_type=jnp.float32) + b_ref[...]
    out_ref[...] = out.astype(out_ref.dtype)


def _gap_head(x, weight, bias):
    N, C, D, H, W = x.shape
    S = D * H * W
    out_c = weight.shape[0]

    tn = N if N % 8 else min(N, 8)
    grid = (pl.cdiv(N, tn),)

    x_nsc = jnp.transpose(x, (0, 2, 3, 4, 1)).reshape(N, S, C)
    b2 = bias.reshape(1, out_c)

    nstream = _NSTREAM if S % _NSTREAM == 0 else 1
    kernel_fn = functools.partial(
        _gap_head_kernel, inv_s=1.0 / S, nstream=nstream)

    sk = S // nstream

    def slice_spec(k):
        return pl.BlockSpec((tn, sk, C), lambda n, k=k: (n, k, 0))

    out_p, fea_p = pl.pallas_call(
        kernel_fn,
        out_shape=(
            jax.ShapeDtypeStruct((N, out_c), x.dtype),
            jax.ShapeDtypeStruct((N, C), x.dtype),
        ),
        grid_spec=pltpu.PrefetchScalarGridSpec(
            num_scalar_prefetch=0,
            grid=grid,
            in_specs=[slice_spec(k) for k in range(nstream)] + [
                pl.BlockSpec((out_c, C), lambda n: (0, 0)),
                pl.BlockSpec((1, out_c), lambda n: (0, 0)),
            ],
            out_specs=(
                pl.BlockSpec((tn, out_c), lambda n: (n, 0)),
                pl.BlockSpec((tn, C), lambda n: (n, 0)),
            ),
        ),
        compiler_params=pltpu.CompilerParams(
            dimension_semantics=("parallel",),
            vmem_limit_bytes=48 << 20,
        ),
        cost_estimate=pl.CostEstimate(
            flops=N * C * S + 2 * N * C * out_c,
            transcendentals=0,
            bytes_accessed=N * C * S * 4 + (C + 1) * out_c * 4
            + N * (out_c + C) * 4,
        ),
    )(*([x_nsc] * nstream), weight, b2)

    return out_p, fea_p


def kernel(x, weight, bias):
    return _gap_head(x, weight, bias)
